# Initial kernel scaffold; baseline (speedup 1.0000x reference)
import jax
import jax.numpy as jnp
from jax.experimental import pallas as pl


def kernel(predicts, objectness, anchors):
    raise NotImplementedError("write your pallas kernel here")


# speedup vs baseline: 14.1628x; 14.1628x over previous
import functools

import jax
import jax.numpy as jnp
from jax.experimental import pallas as pl
from jax.experimental.pallas import tpu as pltpu

_K = 2000
_OUT = 1000
_THR = 0.7
_IMG = 1024.0
_NEG = -1e9
_CHUNK = 200


def _decode_rows(ax1, ay1, ax2, ay2, dx, dy, dw, dh):
    aw = ax2 - ax1
    ah = ay2 - ay1
    acx = ax1 + 0.5 * aw
    acy = ay1 + 0.5 * ah
    pcx = acx + dx * aw
    pcy = acy + dy * ah
    pw = jnp.exp(dw) * aw
    ph = jnp.exp(dh) * ah
    x1 = pcx - 0.5 * pw
    y1 = pcy - 0.5 * ph
    x2 = x1 + pw
    y2 = y1 + ph
    clip = lambda v: jnp.clip(v, 0.0, _IMG)
    return clip(x1), clip(y1), clip(x2), clip(y2)


def _nms_kernel(anch_t_ref, pred_t_ref, anch_c_ref, pred_c_ref, scores_ref,
                boxes_out_ref, scores_out_ref, m_ref):
    at = anch_t_ref[0]
    pt = pred_t_ref[0]
    x1, y1, x2, y2 = _decode_rows(
        at[0:1, :], at[1:2, :], at[2:3, :], at[3:4, :],
        pt[0:1, :], pt[1:2, :], pt[2:3, :], pt[3:4, :])
    area_r = (x2 - x1) * (y2 - y1)

    lane_i = jax.lax.broadcasted_iota(jnp.int32, (1, _K), 1)

    for c in range(_K // _CHUNK):
        r0 = c * _CHUNK
        ac = anch_c_ref[0]
        pc = pred_c_ref[0]
        sl = lambda ref, k: ref[r0:r0 + _CHUNK, k:k + 1]
        cx1, cy1, cx2, cy2 = _decode_rows(
            sl(ac, 0), sl(ac, 1), sl(ac, 2), sl(ac, 3),
            sl(pc, 0), sl(pc, 1), sl(pc, 2), sl(pc, 3))
        area_c = (cx2 - cx1) * (cy2 - cy1)

        w = jnp.maximum(jnp.minimum(cx2, x2) - jnp.maximum(cx1, x1), 0.0)
        h = jnp.maximum(jnp.minimum(cy2, y2) - jnp.maximum(cy1, y1), 0.0)
        inter = w * h
        iou = inter / (area_c + area_r - inter + 1e-9)
        row_i = (jax.lax.broadcasted_iota(jnp.int32, (_CHUNK, 1), 0) + r0)
        m = jnp.where((iou > _THR) & (lane_i > row_i), 1.0, 0.0)
        m_ref[r0:r0 + _CHUNK, :] = m

    def body(i, keep):
        row = m_ref[pl.ds(i, 1), :]
        k_i = jnp.sum(jnp.where(lane_i == i, keep, 0.0))
        return keep * (1.0 - row * k_i)

    keep = jax.lax.fori_loop(0, _K, body,
                             jnp.ones((1, _K), jnp.float32))

    scores = scores_ref[0]
    kept_scores = jnp.where(keep > 0.5, scores, _NEG)

    def cumsum_lanes(v):
        s = 1
        while s < _K:
            v = v + jnp.concatenate(
                [jnp.zeros((1, s), jnp.float32), v[:, :_K - s]], axis=1)
            s *= 2
        return v

    ck = cumsum_lanes(keep)
    cs = cumsum_lanes(1.0 - keep)
    total_kept = jnp.sum(keep)
    dest = jnp.where(keep > 0.5, ck - 1.0, total_kept + cs - 1.0)

    d_iota = jax.lax.broadcasted_iota(jnp.int32, (_OUT, 1), 0)
    perm = jnp.where(dest.astype(jnp.int32) == d_iota, 1.0, 0.0)

    boxes_t = jnp.concatenate([x1, y1, x2, y2], axis=0)
    dot = functools.partial(
        jax.lax.dot_general,
        dimension_numbers=(((1,), (1,)), ((), ())),
        preferred_element_type=jnp.float32)
    boxes_out_ref[0] = dot(perm, boxes_t)
    scores_out_ref[0] = dot(perm, kept_scores)


def kernel(predicts, objectness, anchors):
    bs = predicts.shape[0]
    top_scores, top_idx = jax.lax.top_k(objectness, _K)
    pred_g = jnp.take_along_axis(predicts, top_idx[..., None], axis=1)
    anch_g = jnp.take(anchors, top_idx, axis=0)
    pred_t = jnp.transpose(pred_g, (0, 2, 1))
    anch_t = jnp.transpose(anch_g, (0, 2, 1))
    scores_r = top_scores[:, None, :]

    spec = lambda *shape: pl.BlockSpec((1,) + shape,
                                       lambda b: (b,) + (0,) * len(shape))
    boxes, scores = pl.pallas_call(
        _nms_kernel,
        grid=(bs,),
        in_specs=[spec(4, _K), spec(4, _K), spec(_K, 4), spec(_K, 4),
                  spec(1, _K)],
        out_specs=[spec(_OUT, 4), spec(_OUT, 1)],
        out_shape=[jax.ShapeDtypeStruct((bs, _OUT, 4), jnp.float32),
                   jax.ShapeDtypeStruct((bs, _OUT, 1), jnp.float32)],
        scratch_shapes=[pltpu.VMEM((_K, _K), jnp.float32)],
        compiler_params=pltpu.CompilerParams(
            vmem_limit_bytes=100 * 1024 * 1024),
    )(anch_t, pred_t, anch_g, pred_g, scores_r)

    return jnp.concatenate([boxes, scores], axis=-1)


# speedup vs baseline: 15.4154x; 1.0884x over previous
import functools

import jax
import jax.numpy as jnp
from jax.experimental import pallas as pl
from jax.experimental.pallas import tpu as pltpu

_TOPK = 2000
_K = 2048
_KL = _K // 8
_OUT = 1000
_THR = 0.7
_IMG = 1024.0
_NEG = -1e9
_CHUNK = 256


def _decode_rows(ax1, ay1, ax2, ay2, dx, dy, dw, dh):
    aw = ax2 - ax1
    ah = ay2 - ay1
    acx = ax1 + 0.5 * aw
    acy = ay1 + 0.5 * ah
    pcx = acx + dx * aw
    pcy = acy + dy * ah
    pw = jnp.exp(dw) * aw
    ph = jnp.exp(dh) * ah
    x1 = pcx - 0.5 * pw
    y1 = pcy - 0.5 * ph
    x2 = x1 + pw
    y2 = y1 + ph
    clip = lambda v: jnp.clip(v, 0.0, _IMG)
    return clip(x1), clip(y1), clip(x2), clip(y2)


def _nms_kernel(anch_t_ref, pred_t_ref, anch_c_ref, pred_c_ref, scores_ref,
                boxes_out_ref, scores_out_ref, m_ref):
    at = anch_t_ref[0]
    pt = pred_t_ref[0]
    x1, y1, x2, y2 = _decode_rows(
        at[0:1, :], at[1:2, :], at[2:3, :], at[3:4, :],
        pt[0:1, :], pt[1:2, :], pt[2:3, :], pt[3:4, :])
    area_r = (x2 - x1) * (y2 - y1)

    lane_i = jax.lax.broadcasted_iota(jnp.int32, (1, _K), 1)

    for c in range(_K // _CHUNK):
        r0 = c * _CHUNK
        ac = anch_c_ref[0]
        pc = pred_c_ref[0]
        sl = lambda ref, k: ref[r0:r0 + _CHUNK, k:k + 1]
        cx1, cy1, cx2, cy2 = _decode_rows(
            sl(ac, 0), sl(ac, 1), sl(ac, 2), sl(ac, 3),
            sl(pc, 0), sl(pc, 1), sl(pc, 2), sl(pc, 3))
        area_c = (cx2 - cx1) * (cy2 - cy1)

        w = jnp.maximum(jnp.minimum(cx2, x2) - jnp.maximum(cx1, x1), 0.0)
        h = jnp.maximum(jnp.minimum(cy2, y2) - jnp.maximum(cy1, y1), 0.0)
        inter = w * h
        iou = inter / (area_c + area_r - inter + 1e-9)
        row_i = (jax.lax.broadcasted_iota(jnp.int32, (_CHUNK, 1), 0) + r0)
        m = jnp.where((iou > _THR) & (lane_i > row_i), 1.0, 0.0)
        m_ref[8 * r0:8 * (r0 + _CHUNK), :] = m.reshape(_CHUNK * 8, _KL)

    idx_f = (jax.lax.broadcasted_iota(jnp.int32, (8, _KL), 0) * _KL
             + jax.lax.broadcasted_iota(jnp.int32, (8, _KL), 1))

    def body(i, keep):
        row = m_ref[pl.ds(i * 8, 8), :]
        k_i = jnp.sum(jnp.where(idx_f == i, keep, 0.0))
        return keep * (1.0 - row * k_i)

    keep_f = jax.lax.fori_loop(0, _TOPK, body,
                               jnp.ones((8, _KL), jnp.float32))

    keep = keep_f.reshape(1, _K)
    keep = jnp.where(lane_i < _TOPK, keep, 0.0)
    scores = scores_ref[0]
    kept_scores = jnp.where(keep > 0.5, scores, _NEG)

    def cumsum_lanes(v):
        s = 1
        while s < _K:
            v = v + jnp.concatenate(
                [jnp.zeros((1, s), jnp.float32), v[:, :_K - s]], axis=1)
            s *= 2
        return v

    ck = cumsum_lanes(keep)
    cs = cumsum_lanes(1.0 - keep)
    total_kept = jnp.sum(keep)
    dest = jnp.where(keep > 0.5, ck - 1.0, total_kept + cs - 1.0)

    d_iota = jax.lax.broadcasted_iota(jnp.int32, (_OUT, 1), 0)
    perm = jnp.where(dest.astype(jnp.int32) == d_iota, 1.0, 0.0)

    boxes_t = jnp.concatenate([x1, y1, x2, y2], axis=0)
    dot = functools.partial(
        jax.lax.dot_general,
        dimension_numbers=(((1,), (1,)), ((), ())),
        preferred_element_type=jnp.float32)
    boxes_out_ref[0] = dot(perm, boxes_t)
    scores_out_ref[0] = dot(perm, kept_scores)


def kernel(predicts, objectness, anchors):
    bs = predicts.shape[0]
    top_scores, top_idx = jax.lax.top_k(objectness, _TOPK)
    pred_g = jnp.take_along_axis(predicts, top_idx[..., None], axis=1)
    anch_g = jnp.take(anchors, top_idx, axis=0)
    pad = ((0, 0), (0, _K - _TOPK), (0, 0))
    pred_g = jnp.pad(pred_g, pad)
    anch_g = jnp.pad(anch_g, pad)
    pred_t = jnp.transpose(pred_g, (0, 2, 1))
    anch_t = jnp.transpose(anch_g, (0, 2, 1))
    scores_r = jnp.pad(top_scores, ((0, 0), (0, _K - _TOPK)),
                       constant_values=_NEG)[:, None, :]

    spec = lambda *shape: pl.BlockSpec((1,) + shape,
                                       lambda b: (b,) + (0,) * len(shape))
    boxes, scores = pl.pallas_call(
        _nms_kernel,
        grid=(bs,),
        in_specs=[spec(4, _K), spec(4, _K), spec(_K, 4), spec(_K, 4),
                  spec(1, _K)],
        out_specs=[spec(_OUT, 4), spec(_OUT, 1)],
        out_shape=[jax.ShapeDtypeStruct((bs, _OUT, 4), jnp.float32),
                   jax.ShapeDtypeStruct((bs, _OUT, 1), jnp.float32)],
        scratch_shapes=[pltpu.VMEM((_K * 8, _KL), jnp.float32)],
        compiler_params=pltpu.CompilerParams(
            vmem_limit_bytes=100 * 1024 * 1024),
    )(anch_t, pred_t, anch_g, pred_g, scores_r)

    return jnp.concatenate([boxes, scores], axis=-1)


# speedup vs baseline: 29.8599x; 1.9370x over previous
import functools

import jax
import jax.numpy as jnp
from jax.experimental import pallas as pl
from jax.experimental.pallas import tpu as pltpu

_TOPK = 2000
_K = 2048
_KL = _K // 8
_OUT = 1000
_THR = 0.7
_IMG = 1024.0
_NEG = -1e9
_CHUNK = 256


def _decode_rows(ax1, ay1, ax2, ay2, dx, dy, dw, dh):
    aw = ax2 - ax1
    ah = ay2 - ay1
    acx = ax1 + 0.5 * aw
    acy = ay1 + 0.5 * ah
    pcx = acx + dx * aw
    pcy = acy + dy * ah
    pw = jnp.exp(dw) * aw
    ph = jnp.exp(dh) * ah
    x1 = pcx - 0.5 * pw
    y1 = pcy - 0.5 * ph
    x2 = x1 + pw
    y2 = y1 + ph
    clip = lambda v: jnp.clip(v, 0.0, _IMG)
    return clip(x1), clip(y1), clip(x2), clip(y2)


def _nms_kernel(anch_t_ref, pred_t_ref, anch_c_ref, pred_c_ref, scores_ref,
                boxes_out_ref, scores_out_ref, m_ref):
    at = anch_t_ref[0]
    pt = pred_t_ref[0]
    x1, y1, x2, y2 = _decode_rows(
        at[0:1, :], at[1:2, :], at[2:3, :], at[3:4, :],
        pt[0:1, :], pt[1:2, :], pt[2:3, :], pt[3:4, :])
    area_r = (x2 - x1) * (y2 - y1)

    lane_i = jax.lax.broadcasted_iota(jnp.int32, (1, _K), 1)

    for c in range(_K // _CHUNK):
        r0 = c * _CHUNK
        ac = anch_c_ref[0]
        pc = pred_c_ref[0]
        sl = lambda ref, k: ref[r0:r0 + _CHUNK, k:k + 1]
        cx1, cy1, cx2, cy2 = _decode_rows(
            sl(ac, 0), sl(ac, 1), sl(ac, 2), sl(ac, 3),
            sl(pc, 0), sl(pc, 1), sl(pc, 2), sl(pc, 3))
        area_c = (cx2 - cx1) * (cy2 - cy1)

        w = jnp.maximum(jnp.minimum(cx2, x2) - jnp.maximum(cx1, x1), 0.0)
        h = jnp.maximum(jnp.minimum(cy2, y2) - jnp.maximum(cy1, y1), 0.0)
        inter = w * h
        cond = inter * (1.0 + _THR) > _THR * (area_c + area_r + 1e-9)
        row_i = (jax.lax.broadcasted_iota(jnp.int32, (_CHUNK, 1), 0) + r0)
        m = jnp.where(cond & (lane_i > row_i), 1.0, 0.0)
        m_ref[8 * r0:8 * (r0 + _CHUNK), :] = m.reshape(_CHUNK * 8, _KL)

    idx_f = (jax.lax.broadcasted_iota(jnp.int32, (8, _KL), 0) * _KL
             + jax.lax.broadcasted_iota(jnp.int32, (8, _KL), 1))

    U = 4

    def msum(vec, pos):
        return jnp.sum(jnp.where(idx_f == pos, vec, 0.0))

    def body(g, keep):
        base = g * U
        rows = [m_ref[pl.ds((base + u) * 8, 8), :] for u in range(U)]
        a = [msum(keep, base + u) for u in range(U)]
        mm = {(u, v): msum(rows[u], base + v)
              for u in range(U - 1) for v in range(u + 1, U)}
        k = [a[0]]
        for v in range(1, U):
            kv = a[v]
            for u in range(v):
                kv = kv * (1.0 - mm[(u, v)] * k[u])
            k.append(kv)
        upd = 1.0 - rows[0] * k[0]
        for u in range(1, U):
            upd = upd * (1.0 - rows[u] * k[u])
        return keep * upd

    keep_f = jax.lax.fori_loop(0, _TOPK // U, body,
                               jnp.ones((8, _KL), jnp.float32))

    keep = keep_f.reshape(1, _K)
    keep = jnp.where(lane_i < _TOPK, keep, 0.0)
    scores = scores_ref[0]
    kept_scores = jnp.where(keep > 0.5, scores, _NEG)

    def cumsum_lanes(v):
        s = 1
        while s < _K:
            v = v + jnp.concatenate(
                [jnp.zeros((1, s), jnp.float32), v[:, :_K - s]], axis=1)
            s *= 2
        return v

    ck = cumsum_lanes(keep)
    cs = cumsum_lanes(1.0 - keep)
    total_kept = jnp.sum(keep)
    dest = jnp.where(keep > 0.5, ck - 1.0, total_kept + cs - 1.0)

    d_iota = jax.lax.broadcasted_iota(jnp.int32, (_OUT, 1), 0)
    perm = jnp.where(dest.astype(jnp.int32) == d_iota, 1.0, 0.0)

    boxes_t = jnp.concatenate([x1, y1, x2, y2], axis=0)
    dot = functools.partial(
        jax.lax.dot_general,
        dimension_numbers=(((1,), (1,)), ((), ())),
        preferred_element_type=jnp.float32)
    boxes_out_ref[0] = dot(perm, boxes_t)
    scores_out_ref[0] = dot(perm, kept_scores)


def kernel(predicts, objectness, anchors):
    bs = predicts.shape[0]
    top_scores, top_idx = jax.lax.top_k(objectness, _TOPK)
    pred_g = jnp.take_along_axis(predicts, top_idx[..., None], axis=1)
    anch_g = jnp.take(anchors, top_idx, axis=0)
    pad = ((0, 0), (0, _K - _TOPK), (0, 0))
    pred_g = jnp.pad(pred_g, pad)
    anch_g = jnp.pad(anch_g, pad)
    pred_t = jnp.transpose(pred_g, (0, 2, 1))
    anch_t = jnp.transpose(anch_g, (0, 2, 1))
    scores_r = jnp.pad(top_scores, ((0, 0), (0, _K - _TOPK)),
                       constant_values=_NEG)[:, None, :]

    spec = lambda *shape: pl.BlockSpec((1,) + shape,
                                       lambda b: (b,) + (0,) * len(shape))
    boxes, scores = pl.pallas_call(
        _nms_kernel,
        grid=(bs,),
        in_specs=[spec(4, _K), spec(4, _K), spec(_K, 4), spec(_K, 4),
                  spec(1, _K)],
        out_specs=[spec(_OUT, 4), spec(_OUT, 1)],
        out_shape=[jax.ShapeDtypeStruct((bs, _OUT, 4), jnp.float32),
                   jax.ShapeDtypeStruct((bs, _OUT, 1), jnp.float32)],
        scratch_shapes=[pltpu.VMEM((_K * 8, _KL), jnp.float32)],
        compiler_params=pltpu.CompilerParams(
            vmem_limit_bytes=100 * 1024 * 1024),
    )(anch_t, pred_t, anch_g, pred_g, scores_r)

    return jnp.concatenate([boxes, scores], axis=-1)
